# baseline (speedup 1.0000x reference)
import functools

import jax
import jax.numpy as jnp
from jax import lax
from jax.experimental import pallas as pl
from jax.experimental.pallas import tpu as pltpu
from jax.experimental.pallas import tpu_sc as plsc

_T = 16384
_DIM = 4096
_E = 64
_K = 8
_G = 8
_GS = _E // _G
_TG = 4
_SCALE = 2.5
_BT = 1024

_NC = 2
_NS = 16
_NW = _NC * _NS
_NTOK = _T // _NW
_L = 16
_NCH = _NTOK // _L


def _score_body(x_ref, w_ref, pout_ref):
    x = x_ref[...]
    w = w_ref[...]
    s = lax.dot_general(w, x, (((1,), (1,)), ((), ())),
                        preferred_element_type=jnp.float32)
    m = jnp.max(s, axis=0, keepdims=True)
    e = jnp.exp(s - m)
    pout_ref[...] = e / jnp.sum(e, axis=0, keepdims=True)


def _tc_score(x, weight):
    return pl.pallas_call(
        _score_body,
        grid=(_T // _BT,),
        in_specs=[
            pl.BlockSpec((_BT, _DIM), lambda i: (i, 0)),
            pl.BlockSpec((_E, _DIM), lambda i: (0, 0)),
        ],
        out_specs=pl.BlockSpec((_E, _BT), lambda i: (0, i)),
        out_shape=jax.ShapeDtypeStruct((_E, _T), jnp.float32),
        compiler_params=pltpu.CompilerParams(
            dimension_semantics=("arbitrary",),
        ),
    )(x, weight)


@functools.partial(
    pl.kernel,
    out_type=[
        jax.ShapeDtypeStruct((_K, _T), jnp.float32),
        jax.ShapeDtypeStruct((_K, _T), jnp.int32),
    ],
    mesh=plsc.VectorSubcoreMesh(core_axis_name="c", subcore_axis_name="s"),
    scratch_types=[
        pltpu.VMEM((_E, _NTOK), jnp.float32),
        pltpu.VMEM((_E, _L), jnp.float32),
        pltpu.VMEM((_K, _NTOK), jnp.float32),
        pltpu.VMEM((_K, _NTOK), jnp.int32),
    ],
)
def _sc_route(probs_hbm, biasb_hbm, wout_hbm, iout_hbm,
              probs_v, bias_v, wv, iv):
    wid = lax.axis_index("s") * _NC + lax.axis_index("c")
    base = wid * _NTOK
    pltpu.sync_copy(probs_hbm.at[:, pl.ds(base, _NTOK)], probs_v)
    pltpu.sync_copy(biasb_hbm, bias_v)

    neg_inf = jnp.full((_L,), -jnp.inf, jnp.float32)

    def chunk(c, carry):
        off = c * _L
        p = [probs_v[e, pl.ds(off, _L)] for e in range(_E)]
        v = [p[e] + bias_v[e, :] for e in range(_E)]

        gs = []
        for g in range(_G):
            grp = v[g * _GS:(g + 1) * _GS]
            m1 = jnp.maximum(grp[0], grp[1])
            m2 = jnp.minimum(grp[0], grp[1])
            for t in grp[2:]:
                m2 = jnp.maximum(m2, jnp.minimum(m1, t))
                m1 = jnp.maximum(m1, t)
            gs.append(m1 + m2)

        sel = [None] * _G
        for _ in range(_TG):
            bestv = gs[0]
            besti = jnp.full((_L,), 0, jnp.int32)
            for g in range(1, _G):
                gt = gs[g] > bestv
                bestv = jnp.where(gt, gs[g], bestv)
                besti = jnp.where(gt, jnp.full((_L,), g, jnp.int32), besti)
            for g in range(_G):
                hit = besti == jnp.full((_L,), g, jnp.int32)
                sel[g] = hit if sel[g] is None else (sel[g] | hit)
                gs[g] = jnp.where(hit, neg_inf, gs[g])

        mv = [jnp.where(sel[e // _GS], v[e], neg_inf) for e in range(_E)]

        for r in range(_K):
            bestv = mv[0]
            besti = jnp.full((_L,), 0, jnp.int32)
            for e in range(1, _E):
                gt = mv[e] > bestv
                bestv = jnp.where(gt, mv[e], bestv)
                besti = jnp.where(gt, jnp.full((_L,), e, jnp.int32), besti)
            wsel = jnp.full((_L,), 0.0, jnp.float32)
            for e in range(_E):
                hit = besti == jnp.full((_L,), e, jnp.int32)
                wsel = jnp.where(hit, p[e], wsel)
                mv[e] = jnp.where(hit, neg_inf, mv[e])
            wv[r, pl.ds(off, _L)] = wsel * jnp.float32(_SCALE)
            iv[r, pl.ds(off, _L)] = besti
        return carry

    lax.fori_loop(0, _NCH, chunk, 0)

    pltpu.sync_copy(wv, wout_hbm.at[:, pl.ds(base, _NTOK)])
    pltpu.sync_copy(iv, iout_hbm.at[:, pl.ds(base, _NTOK)])


@jax.jit
def kernel(x, weight, bias):
    probs_t = _tc_score(x, weight)
    biasb = jnp.broadcast_to(bias.reshape(_E, 1), (_E, _L))
    wt, it = _sc_route(probs_t, biasb)
    return wt.T.astype(x.dtype), it.T


# speedup vs baseline: 2.5044x; 2.5044x over previous
import functools

import jax
import jax.numpy as jnp
from jax import lax
from jax.experimental import pallas as pl
from jax.experimental.pallas import tpu as pltpu

_T = 16384
_DIM = 4096
_E = 64
_K = 8
_G = 8
_GS = _E // _G
_TG = 4
_SCALE = 2.5
_BT = 1024


def _gate_body(xa_ref, xb_ref, wa_ref, wb_ref, b_ref, wout_ref, iout_ref):
    s = lax.dot_general(wa_ref[...], xa_ref[...], (((1,), (1,)), ((), ())),
                        preferred_element_type=jnp.float32)
    s = s + lax.dot_general(wb_ref[...], xb_ref[...], (((1,), (1,)), ((), ())),
                            preferred_element_type=jnp.float32)
    neg_inf = jnp.float32(-jnp.inf)

    m = jnp.max(s, axis=0, keepdims=True)
    e = jnp.exp(s - m)
    probs = e / jnp.sum(e, axis=0, keepdims=True)
    biased = probs + b_ref[...]

    gscore_rows = []
    for g in range(_G):
        grp = biased[g * _GS:(g + 1) * _GS, :]
        m1 = jnp.max(grp, axis=0, keepdims=True)
        eq = grp == m1
        cnt = jnp.sum(eq.astype(jnp.float32), axis=0, keepdims=True)
        m2 = jnp.max(jnp.where(eq, neg_inf, grp), axis=0, keepdims=True)
        m2 = jnp.where(cnt > 1.5, m1, m2)
        gscore_rows.append(m1 + m2)
    gscore = jnp.concatenate(gscore_rows, axis=0)

    riota_g = lax.broadcasted_iota(jnp.int32, (_G, _BT), 0)
    sel = jnp.zeros((_G, _BT), dtype=jnp.bool_)
    gs = gscore
    for _ in range(_TG):
        mx = jnp.max(gs, axis=0, keepdims=True)
        a = jnp.min(jnp.where(gs == mx, riota_g, _E), axis=0, keepdims=True)
        hit = riota_g == a
        sel = sel | hit
        gs = jnp.where(hit, neg_inf, gs)

    ms_rows = []
    for g in range(_G):
        grp = biased[g * _GS:(g + 1) * _GS, :]
        ms_rows.append(jnp.where(sel[g:g + 1, :], grp, neg_inf))
    ms = jnp.concatenate(ms_rows, axis=0)

    riota_e = lax.broadcasted_iota(jnp.int32, (_E, _BT), 0)
    idx_rows = []
    w_rows = []
    for _ in range(_K):
        mx = jnp.max(ms, axis=0, keepdims=True)
        a = jnp.min(jnp.where(ms == mx, riota_e, _E), axis=0, keepdims=True)
        hit = riota_e == a
        wv = jnp.sum(jnp.where(hit, probs, 0.0), axis=0, keepdims=True)
        idx_rows.append(a)
        w_rows.append(wv)
        ms = jnp.where(hit, neg_inf, ms)

    wout_ref[...] = jnp.concatenate(w_rows, axis=0) * jnp.float32(_SCALE)
    iout_ref[...] = jnp.concatenate(idx_rows, axis=0)


@jax.jit
def kernel(x, weight, bias):
    bias2 = bias.reshape(_E, 1)
    grid = (_T // _BT,)
    wt, it = pl.pallas_call(
        _gate_body,
        grid=grid,
        in_specs=[
            pl.BlockSpec((_BT, _DIM // 2), lambda i: (i, 0)),
            pl.BlockSpec((_BT, _DIM // 2), lambda i: (i, 1)),
            pl.BlockSpec((_E, _DIM // 2), lambda i: (0, 0)),
            pl.BlockSpec((_E, _DIM // 2), lambda i: (0, 1)),
            pl.BlockSpec((_E, 1), lambda i: (0, 0)),
        ],
        out_specs=[
            pl.BlockSpec((_K, _BT), lambda i: (0, i)),
            pl.BlockSpec((_K, _BT), lambda i: (0, i)),
        ],
        out_shape=[
            jax.ShapeDtypeStruct((_K, _T), jnp.float32),
            jax.ShapeDtypeStruct((_K, _T), jnp.int32),
        ],
        compiler_params=pltpu.CompilerParams(
            dimension_semantics=("parallel",),
            vmem_limit_bytes=100 * 1024 * 1024,
        ),
    )(x, x, weight, weight, bias2)
    return wt.T.astype(x.dtype), it.T


# speedup vs baseline: 2.5380x; 1.0134x over previous
import functools

import jax
import jax.numpy as jnp
from jax import lax
from jax.experimental import pallas as pl
from jax.experimental.pallas import tpu as pltpu

_T = 16384
_DIM = 4096
_E = 64
_K = 8
_G = 8
_GS = _E // _G
_TG = 4
_SCALE = 2.5
_BT = 1024


def _gate_body(x_ref, w_ref, b_ref, wout_ref, iout_ref):
    x = x_ref[...]
    w = w_ref[...]
    s = lax.dot_general(w, x, (((1,), (1,)), ((), ())),
                        preferred_element_type=jnp.float32)
    neg_inf = jnp.float32(-jnp.inf)

    m = jnp.max(s, axis=0, keepdims=True)
    e = jnp.exp(s - m)
    probs = e / jnp.sum(e, axis=0, keepdims=True)
    biased = probs + b_ref[...]

    gscore_rows = []
    for g in range(_G):
        grp = biased[g * _GS:(g + 1) * _GS, :]
        m1 = jnp.max(grp, axis=0, keepdims=True)
        eq = grp == m1
        cnt = jnp.sum(eq.astype(jnp.float32), axis=0, keepdims=True)
        m2 = jnp.max(jnp.where(eq, neg_inf, grp), axis=0, keepdims=True)
        m2 = jnp.where(cnt > 1.5, m1, m2)
        gscore_rows.append(m1 + m2)
    gscore = jnp.concatenate(gscore_rows, axis=0)

    riota_g = lax.broadcasted_iota(jnp.int32, (_G, _BT), 0)
    sel = jnp.zeros((_G, _BT), dtype=jnp.bool_)
    gs = gscore
    for _ in range(_TG):
        mx = jnp.max(gs, axis=0, keepdims=True)
        a = jnp.min(jnp.where(gs == mx, riota_g, _E), axis=0, keepdims=True)
        hit = riota_g == a
        sel = sel | hit
        gs = jnp.where(hit, neg_inf, gs)

    ms_rows = []
    for g in range(_G):
        grp = biased[g * _GS:(g + 1) * _GS, :]
        ms_rows.append(jnp.where(sel[g:g + 1, :], grp, neg_inf))
    ms = jnp.concatenate(ms_rows, axis=0)

    riota_e = lax.broadcasted_iota(jnp.int32, (_E, _BT), 0)
    idx_rows = []
    w_rows = []
    for _ in range(_K):
        mx = jnp.max(ms, axis=0, keepdims=True)
        a = jnp.min(jnp.where(ms == mx, riota_e, _E), axis=0, keepdims=True)
        hit = riota_e == a
        wv = jnp.sum(jnp.where(hit, probs, 0.0), axis=0, keepdims=True)
        idx_rows.append(a)
        w_rows.append(wv)
        ms = jnp.where(hit, neg_inf, ms)

    wout_ref[...] = jnp.concatenate(w_rows, axis=0) * jnp.float32(_SCALE)
    iout_ref[...] = jnp.concatenate(idx_rows, axis=0)


@jax.jit
def kernel(x, weight, bias):
    bias2 = bias.reshape(_E, 1)
    grid = (_T // _BT,)
    wt, it = pl.pallas_call(
        _gate_body,
        grid=grid,
        in_specs=[
            pl.BlockSpec((_BT, _DIM), lambda i: (i, 0)),
            pl.BlockSpec((_E, _DIM), lambda i: (0, 0)),
            pl.BlockSpec((_E, 1), lambda i: (0, 0)),
        ],
        out_specs=[
            pl.BlockSpec((_K, _BT), lambda i: (0, i)),
            pl.BlockSpec((_K, _BT), lambda i: (0, i)),
        ],
        out_shape=[
            jax.ShapeDtypeStruct((_K, _T), jnp.float32),
            jax.ShapeDtypeStruct((_K, _T), jnp.int32),
        ],
        compiler_params=pltpu.CompilerParams(
            dimension_semantics=("parallel",),
            vmem_limit_bytes=100 * 1024 * 1024,
        ),
    )(x, weight, bias2)
    return wt.T.astype(x.dtype), it.T
